# baseline (speedup 1.0000x reference)
import jax
import jax.numpy as jnp
from jax import lax
from jax.experimental import pallas as pl
from jax.experimental.pallas import tpu as pltpu
from jax.experimental.pallas import tpu_sc as plsc

_NC = 2
_NS = 16
_L = 16
_NW = _NC * _NS

_NB = 128


def _sc_lorentz_products(table, idx_i, ks2d, B, N, D):
    V = table.shape[0]
    del V
    bpw = B // _NW
    nsub = bpw // _NB
    rows = _NB * N
    ngrp = rows // _L
    krows = rows // 128

    mesh = plsc.VectorSubcoreMesh(core_axis_name="c", subcore_axis_name="s")

    @pl.kernel(
        out_type=jax.ShapeDtypeStruct((B * N,), jnp.float32),
        mesh=mesh,
        scratch_types=[
            pltpu.VMEM((_NB,), jnp.int32),
            pltpu.VMEM((krows, 128), jnp.int32),
            pltpu.VMEM((_NB, D), jnp.float32),
            pltpu.VMEM((rows, D), jnp.float32),
            pltpu.VMEM((rows,), jnp.float32),
            pltpu.SemaphoreType.DMA,
        ],
        compiler_params=pltpu.CompilerParams(
            needs_layout_passes=False, use_tc_tiling_on_sc=False),
    )
    def sc_kernel(table_hbm, i_hbm, k_hbm, out_hbm,
                  iv, kv, uiv, ukv, outv, sem):
        wid = lax.axis_index("s") * _NC + lax.axis_index("c")
        for sub in range(nsub):
            base_b = wid * bpw + sub * _NB
            pltpu.sync_copy(i_hbm.at[pl.ds(base_b, _NB)], iv)
            pltpu.sync_copy(k_hbm.at[wid * nsub + sub], kv)
            copies = [pltpu.async_copy(table_hbm.at[iv], uiv, sem)]
            for j in range(krows):
                copies.append(pltpu.async_copy(
                    table_hbm.at[kv.at[j]],
                    ukv.at[pl.ds(j * 128, 128)], sem))
            for c in copies:
                c.wait()

            @pl.loop(0, ngrp)
            def _(g):
                p0 = g * _L
                pair = p0 + lax.iota(jnp.int32, _L)
                brow = pair // N
                col = jnp.zeros((_L,), jnp.int32)
                vk = plsc.load_gather(ukv, [pair, col])
                vu = plsc.load_gather(uiv, [brow, col])
                acc = -(vk * vu)
                for d0 in range(1, D):
                    cold = jnp.full((_L,), d0, jnp.int32)
                    vk = plsc.load_gather(ukv, [pair, cold])
                    vu = plsc.load_gather(uiv, [brow, cold])
                    acc = acc + vk * vu
                outv[pl.ds(p0, _L)] = acc

            pltpu.sync_copy(outv, out_hbm.at[pl.ds(base_b * N, rows)])

    return sc_kernel(table, idx_i, ks2d)


def _tc_tail(lsp2d):
    Bn = lsp2d.shape[0]

    def body(lsp_ref, out_ref):
        dd = -lsp_ref[...]
        dd = jnp.where(dd <= 1.0, jnp.float32(1.0 + 1e-6), dd)
        dd = -jnp.log(dd + jnp.sqrt(dd * dd - 1.0))
        lse = jnp.log(jnp.sum(jnp.exp(dd), axis=1) + 1e-6)
        out_ref[...] = lse - dd[:, 0]

    return pl.pallas_call(
        body,
        out_shape=jax.ShapeDtypeStruct((Bn,), jnp.float32),
    )(lsp2d)


def kernel(table, I, Ks):
    B, N = Ks.shape
    D = table.shape[1]
    nchunks = B // _NB
    ks3d = Ks.reshape(nchunks, _NB * N // 128, 128).astype(jnp.int32)
    lsp = _sc_lorentz_products(table, I.astype(jnp.int32), ks3d, B, N, D)
    return _tc_tail(lsp.reshape(B, N))


# speedup vs baseline: 1.0226x; 1.0226x over previous
import jax
import jax.numpy as jnp
from jax import lax
from jax.experimental import pallas as pl
from jax.experimental.pallas import tpu as pltpu
from jax.experimental.pallas import tpu_sc as plsc

_NC = 2
_NS = 16
_L = 16
_NW = _NC * _NS

_NB = 32


def _sc_lorentz_products(t128, idx_i, ks3d, B, N, D):
    bpw = B // _NW
    nsub = bpw // _NB
    rows = _NB * N
    ngrp = rows // _L
    krows = rows // 128

    mesh = plsc.VectorSubcoreMesh(core_axis_name="c", subcore_axis_name="s")

    @pl.kernel(
        out_type=jax.ShapeDtypeStruct((B * N,), jnp.float32),
        mesh=mesh,
        scratch_types=[
            pltpu.VMEM((_NB,), jnp.int32),
            pltpu.VMEM((_NB,), jnp.int32),
            pltpu.VMEM((_NB,), jnp.int32),
            pltpu.VMEM((krows, 128), jnp.int32),
            pltpu.VMEM((krows, 128), jnp.int32),
            pltpu.VMEM((krows, 128), jnp.int32),
            pltpu.VMEM((_NB, 128), jnp.float32),
            pltpu.VMEM((rows, 128), jnp.float32),
            pltpu.VMEM((rows,), jnp.float32),
            pltpu.SemaphoreType.DMA,
        ],
        compiler_params=pltpu.CompilerParams(
            needs_layout_passes=False, use_tc_tiling_on_sc=False),
    )
    def sc_kernel(table_hbm, i_hbm, k_hbm, out_hbm,
                  iv, ivq, ivo, kv, kvq, kvo, uiv, ukv, outv, sem):
        wid = lax.axis_index("s") * _NC + lax.axis_index("c")

        @pl.loop(0, nsub)
        def _(sub):
            base_b = wid * bpw + sub * _NB
            pltpu.sync_copy(i_hbm.at[pl.ds(base_b, _NB)], iv)
            pltpu.sync_copy(k_hbm.at[wid * nsub + sub], kv)
            for j0 in range(0, _NB, _L):
                v = iv[pl.ds(j0, _L)]
                ivq[pl.ds(j0, _L)] = v >> 2
                ivo[pl.ds(j0, _L)] = (v & 3) * D
            for j in range(krows):
                for j0 in range(0, 128, _L):
                    v = kv[j, pl.ds(j0, _L)]
                    kvq[j, pl.ds(j0, _L)] = v >> 2
                    kvo[j, pl.ds(j0, _L)] = (v & 3) * D
            copies = [pltpu.async_copy(table_hbm.at[ivq], uiv, sem)]
            for j in range(krows):
                copies.append(pltpu.async_copy(
                    table_hbm.at[kvq.at[j]],
                    ukv.at[pl.ds(j * 128, 128)], sem))
            for c in copies:
                c.wait()

            @pl.loop(0, ngrp)
            def _(g):
                p0 = g * _L
                pair = p0 + lax.iota(jnp.int32, _L)
                brow = pair // N
                koffv = plsc.load_gather(kvo, [pair >> 7, pair & 127])
                ioffv = plsc.load_gather(ivo, [brow])
                vk = plsc.load_gather(ukv, [pair, koffv])
                vu = plsc.load_gather(uiv, [brow, ioffv])
                acc = -(vk * vu)
                for d0 in range(1, D):
                    vk = plsc.load_gather(ukv, [pair, koffv + d0])
                    vu = plsc.load_gather(uiv, [brow, ioffv + d0])
                    acc = acc + vk * vu
                outv[pl.ds(p0, _L)] = acc

            pltpu.sync_copy(outv, out_hbm.at[pl.ds(base_b * N, rows)])

    return sc_kernel(t128, idx_i, ks3d)


def _tc_pack(tabT):
    D, V = tabT.shape
    blocks = (V + 4095) // 4096
    R = blocks * 1024

    def body(x_ref, o_ref):
        y = jnp.transpose(x_ref[...])
        y2 = y.reshape(1024, 4, D)
        for jj in range(4):
            o_ref[:, jj * D:(jj + 1) * D] = y2[:, jj, :]

    return pl.pallas_call(
        body,
        grid=(blocks,),
        in_specs=[pl.BlockSpec((D, 4096), lambda i: (0, i))],
        out_specs=pl.BlockSpec((1024, 4 * D), lambda i: (i, 0)),
        out_shape=jax.ShapeDtypeStruct((R, 4 * D), jnp.float32),
    )(tabT)


def _tc_tail(lsp2d):
    Bn = lsp2d.shape[0]

    def body(lsp_ref, out_ref):
        dd = -lsp_ref[...]
        dd = jnp.where(dd <= 1.0, jnp.float32(1.0 + 1e-6), dd)
        dd = -jnp.log(dd + jnp.sqrt(dd * dd - 1.0))
        lse = jnp.log(jnp.sum(jnp.exp(dd), axis=1) + 1e-6)
        out_ref[...] = lse - dd[:, 0]

    return pl.pallas_call(
        body,
        out_shape=jax.ShapeDtypeStruct((Bn,), jnp.float32),
    )(lsp2d)


def kernel(table, I, Ks):
    B, N = Ks.shape
    V, D = table.shape
    t128 = _tc_pack(jnp.transpose(table))
    nchunks = B // _NB
    ks3d = Ks.reshape(nchunks, _NB * N // 128, 128).astype(jnp.int32)
    lsp = _sc_lorentz_products(t128, I.astype(jnp.int32), ks3d, B, N, D)
    return _tc_tail(lsp.reshape(B, N))


# speedup vs baseline: 1.1643x; 1.1386x over previous
import jax
import jax.numpy as jnp
from jax import lax
from jax.experimental import pallas as pl
from jax.experimental.pallas import tpu as pltpu
from jax.experimental.pallas import tpu_sc as plsc

_NC = 2
_NS = 16
_L = 16
_NW = _NC * _NS

_NB = 64


def _sc_lorentz_products(t32, idx_i, ks3d, B, N, D):
    bpw = B // _NW
    nsub = bpw // _NB
    rows = _NB * N
    ngrp = rows // _L
    krows = rows // 128
    kpw = bpw * N // 128

    mesh = plsc.VectorSubcoreMesh(core_axis_name="c", subcore_axis_name="s")

    @pl.kernel(
        out_type=jax.ShapeDtypeStruct((B * N,), jnp.float32),
        mesh=mesh,
        scratch_types=[
            pltpu.VMEM((bpw,), jnp.int32),
            pltpu.VMEM((kpw, 128), jnp.int32),
            pltpu.VMEM((2, _NB, D), jnp.float32),
            pltpu.VMEM((2, rows, D), jnp.float32),
            pltpu.VMEM((2, rows), jnp.float32),
            pltpu.SemaphoreType.DMA,
            pltpu.SemaphoreType.DMA,
        ],
        compiler_params=pltpu.CompilerParams(
            needs_layout_passes=False, use_tc_tiling_on_sc=False),
    )
    def sc_kernel(table_hbm, i_hbm, k_hbm, out_hbm,
                  iv, kv, uiv, ukv, outv, gsem, wsem):
        wid = lax.axis_index("s") * _NC + lax.axis_index("c")
        base_b = wid * bpw
        pltpu.sync_copy(i_hbm.at[pl.ds(base_b, bpw)], iv)
        pltpu.sync_copy(k_hbm.at[wid], kv)

        def fire(s):
            buf = s % 2
            cs = [pltpu.async_copy(
                table_hbm.at[iv.at[pl.ds(s * _NB, _NB)]],
                uiv.at[buf], gsem)]
            for j in range(krows):
                cs.append(pltpu.async_copy(
                    table_hbm.at[kv.at[s * krows + j]],
                    ukv.at[buf].at[pl.ds(j * 128, 128)], gsem))
            return cs

        pend = {0: fire(0), 1: fire(1)}
        wb = {}
        for s in range(nsub):
            buf = s % 2
            for c in pend.pop(s):
                c.wait()
            if s - 2 in wb:
                wb.pop(s - 2).wait()
            ukb = ukv.at[buf]
            uib = uiv.at[buf]
            outb = outv.at[buf]

            @pl.loop(0, ngrp)
            def _(g):
                p0 = g * _L
                pair = p0 + lax.iota(jnp.int32, _L)
                brow = pair // N
                c0 = jnp.zeros((_L,), jnp.int32)
                a0 = -(plsc.load_gather(ukb, [pair, c0])
                       * plsc.load_gather(uib, [brow, c0]))
                a1 = jnp.zeros((_L,), jnp.float32)
                a2 = jnp.zeros((_L,), jnp.float32)
                a3 = jnp.zeros((_L,), jnp.float32)
                accs = [a0, a1, a2, a3]
                for d0 in range(1, D):
                    cd = jnp.full((_L,), d0, jnp.int32)
                    prod = (plsc.load_gather(ukb, [pair, cd])
                            * plsc.load_gather(uib, [brow, cd]))
                    accs[d0 % 4] = accs[d0 % 4] + prod
                outb[pl.ds(p0, _L)] = ((accs[0] + accs[1])
                                       + (accs[2] + accs[3]))

            wb[s] = pltpu.async_copy(
                outb, out_hbm.at[pl.ds((base_b + s * _NB) * N, rows)], wsem)
            if s + 2 < nsub:
                pend[s + 2] = fire(s + 2)
        for s in sorted(wb):
            wb.pop(s).wait()

    return sc_kernel(t32, idx_i, ks3d)


def _tc_pack(tabT):
    D, V = tabT.shape
    blocks = (V + 4095) // 4096

    def body(x_ref, o_ref):
        y = jnp.transpose(x_ref[...])
        y2 = y.reshape(1024, 4, D)
        for jj in range(4):
            o_ref[:, jj * D:(jj + 1) * D] = y2[:, jj, :]

    return pl.pallas_call(
        body,
        grid=(blocks,),
        in_specs=[pl.BlockSpec((D, 4096), lambda i: (0, i))],
        out_specs=pl.BlockSpec((1024, 4 * D), lambda i: (i, 0)),
        out_shape=jax.ShapeDtypeStruct((blocks * 1024, 4 * D), jnp.float32),
    )(tabT)


def _tc_tail(lsp2d):
    Bn = lsp2d.shape[0]

    def body(lsp_ref, out_ref):
        dd = -lsp_ref[...]
        dd = jnp.where(dd <= 1.0, jnp.float32(1.0 + 1e-6), dd)
        dd = -jnp.log(dd + jnp.sqrt(dd * dd - 1.0))
        lse = jnp.log(jnp.sum(jnp.exp(dd), axis=1) + 1e-6)
        out_ref[...] = lse - dd[:, 0]

    return pl.pallas_call(
        body,
        out_shape=jax.ShapeDtypeStruct((Bn,), jnp.float32),
    )(lsp2d)


def kernel(table, I, Ks):
    B, N = Ks.shape
    V, D = table.shape
    t128 = _tc_pack(jnp.transpose(table))
    t32 = t128.reshape(t128.shape[0] * 4, D)
    ks3d = Ks.reshape(_NW, B * N // (128 * _NW), 128).astype(jnp.int32)
    lsp = _sc_lorentz_products(t32, I.astype(jnp.int32), ks3d, B, N, D)
    return _tc_tail(lsp.reshape(B, N))
